# baseline (speedup 1.0000x reference)
import functools

import jax
import jax.numpy as jnp
from jax import lax
from jax.experimental import pallas as pl
from jax.experimental.pallas import tpu as pltpu
from jax.experimental.pallas import tpu_sc as plsc

_N, _E, _I, _D = 10000, 320000, 128, 3
_NC, _NS = 2, 16
_NW = _NC * _NS
_EPW = _E // _NW
_C = 80
_NCH = _EPW // _C
_UROWS = 80
_NU = _N // _UROWS


def _prep_body(coef_ref, x_ref, w_ref, dst_ref, src_ref,
               table_ref, idx_ref, srcf_ref):
    table_ref[...] = x_ref[...] * coef_ref[0]

    @pl.when(pl.program_id(0) == 0)
    def _():
        w = w_ref[...]
        idx_ref[...] = jnp.where(w > 0, (w - 1) * _N + dst_ref[...], 0)
        rows = lax.broadcasted_iota(jnp.int32, w.shape, 0)
        cols = lax.broadcasted_iota(jnp.int32, w.shape, 1)
        trash = _N + lax.rem(rows * _I + cols, _UROWS)
        srcf_ref[...] = jnp.where(w > 0, src_ref[...], trash)


def _build_table_and_idx(x, coefs, w2d, dst2d, src2d):
    return pl.pallas_call(
        _prep_body,
        grid=(_D,),
        in_specs=[
            pl.BlockSpec((1, 1, _I), lambda d: (d, 0, 0)),
            pl.BlockSpec((_N, _I), lambda d: (0, 0)),
            pl.BlockSpec(w2d.shape, lambda d: (0, 0)),
            pl.BlockSpec(w2d.shape, lambda d: (0, 0)),
            pl.BlockSpec(w2d.shape, lambda d: (0, 0)),
        ],
        out_specs=[
            pl.BlockSpec((_N, _I), lambda d: (d, 0)),
            pl.BlockSpec(w2d.shape, lambda d: (0, 0)),
            pl.BlockSpec(w2d.shape, lambda d: (0, 0)),
        ],
        out_shape=[
            jax.ShapeDtypeStruct((_D * _N, _I), jnp.float32),
            jax.ShapeDtypeStruct(w2d.shape, jnp.int32),
            jax.ShapeDtypeStruct(w2d.shape, jnp.int32),
        ],
    )(coefs, x, w2d, dst2d, src2d)


def _sc_aggregate(table, idx, src):
    mesh = plsc.VectorSubcoreMesh(core_axis_name="c", subcore_axis_name="s")

    @functools.partial(
        pl.kernel,
        out_type=jax.ShapeDtypeStruct((_NC * _N, _I), jnp.float32),
        mesh=mesh,
        scratch_types=[
            pltpu.VMEM((_EPW,), jnp.int32),
            pltpu.VMEM((_C,), jnp.int32),
            pltpu.VMEM((_C,), jnp.int32),
            pltpu.VMEM((_C,), jnp.int32),
            pltpu.VMEM((_C, _I), jnp.float32),
            pltpu.VMEM((_C, _I), jnp.float32),
            pltpu.VMEM((_C, _I), jnp.float32),
            pltpu.VMEM((_UROWS // 2, _I), jnp.float32),
            pltpu.VMEM_SHARED((_N + _UROWS, _I), jnp.float32),
            pltpu.SemaphoreType.DMA,
            pltpu.SemaphoreType.DMA,
            pltpu.SemaphoreType.DMA,
            pltpu.SemaphoreType.DMA,
            pltpu.SemaphoreType.DMA,
            pltpu.SemaphoreType.DMA,
            pltpu.SemaphoreType.DMA,
        ],
    )
    def body(table_hbm, idx_hbm, src_hbm, out_hbm,
             idx_buf, src_0, src_1, src_2, bank_0, bank_1, bank_2, zbuf,
             accum, fsem_0, fsem_1, fsem_2, ssem_0, ssem_1, ssem_2, isem):
        srcs = (src_0, src_1, src_2)
        banks = (bank_0, bank_1, bank_2)
        fsems = (fsem_0, fsem_1, fsem_2)
        ssems = (ssem_0, ssem_1, ssem_2)
        c = lax.axis_index("c")
        s = lax.axis_index("s")
        wid = s * _NC + c
        base = pl.multiple_of(wid * _EPW, 8)
        pltpu.sync_copy(idx_hbm.at[pl.ds(base, _EPW)], idx_buf)

        n_units = (_NU - 1 - s) // _NS + 1

        def init_unit(k, carry):
            r = pl.multiple_of((s + k * _NS) * _UROWS, 8)
            lo = pltpu.make_async_copy(
                zbuf, accum.at[pl.ds(r, _UROWS // 2)], isem)
            hi = pltpu.make_async_copy(
                zbuf, accum.at[pl.ds(r + _UROWS // 2, _UROWS // 2)], isem)
            return lo, hi

        def fetch(ch, r):
            off = pl.multiple_of(ch * _C, 8)
            rows = pltpu.make_async_copy(
                table_hbm.at[idx_buf.at[pl.ds(off, _C)]], banks[r], fsems[r])
            sidx = pltpu.make_async_copy(
                src_hbm.at[pl.ds(base + off, _C)], srcs[r], fsems[r])
            return rows, sidx

        def scat(r):
            return pltpu.make_async_copy(banks[r], accum.at[srcs[r]], ssems[r])

        for cp in fetch(0, 0) + fetch(1, 1):
            cp.start()

        def zero_row(i, carry):
            for j in range(_I // 16):
                zbuf[i, pl.ds(j * 16, 16)] = jnp.zeros((16,), jnp.float32)
            return carry

        lax.fori_loop(0, _UROWS // 2, zero_row, 0)

        def start_init(k, carry):
            for cp in init_unit(k, carry):
                cp.start()
            return carry

        def wait_init(k, carry):
            for cp in init_unit(k, carry):
                cp.wait()
            return carry

        lax.fori_loop(0, n_units, start_init, 0)
        lax.fori_loop(0, n_units, wait_init, 0)
        plsc.subcore_barrier()

        def run_chunk(ch, r):
            t = (r + 2) % 3

            @pl.when((ch >= 1) & (ch < _NCH - 2))
            def _():
                scat(t).wait()

            @pl.when(ch < _NCH - 2)
            def _():
                for cp in fetch(ch + 2, t):
                    cp.start()

            for cp in fetch(ch, r):
                cp.wait()
            scat(r).start(add=True)

        def triple_body(g, carry):
            for r in range(3):
                run_chunk(g * 3 + r, r)
            return carry

        lax.fori_loop(0, _NCH // 3, triple_body, 0)
        for ch in range(_NCH - _NCH % 3, _NCH):
            for cp in fetch(ch, ch % 3):
                cp.wait()
            scat(ch % 3).start(add=True)
        for r in ((_NCH - 3) % 3, (_NCH - 2) % 3, (_NCH - 1) % 3):
            scat(r).wait()
        plsc.subcore_barrier()

        def write_unit(k, carry):
            r = pl.multiple_of((s + k * _NS) * _UROWS, 8)
            return pltpu.make_async_copy(accum.at[pl.ds(r, _UROWS)],
                                         out_hbm.at[pl.ds(c * _N + r, _UROWS)],
                                         isem)

        lax.fori_loop(0, n_units,
                      lambda k, c: (write_unit(k, c).start(), c)[1], 0)
        lax.fori_loop(0, n_units,
                      lambda k, c: (write_unit(k, c).wait(), c)[1], 0)

    return body(table, idx, src)


def _mlp_body(x_ref, parts_ref, w1_ref, b1_ref, g1_ref, be1_ref,
              w2_ref, b2_ref, g2_ref, be2_ref, out_ref):
    combined = x_ref[...] + parts_ref[0] + parts_ref[1]

    def layer(h, w_ref, b_ref, g_ref, be_ref):
        h = lax.dot_general(h, w_ref[...], (((1,), (1,)), ((), ())),
                            preferred_element_type=jnp.float32)
        h = h + b_ref[...]
        mu = jnp.mean(h, axis=0, keepdims=True)
        var = jnp.mean((h - mu) ** 2, axis=0, keepdims=True)
        h = g_ref[...] * (h - mu) / jnp.sqrt(var + 1e-5) + be_ref[...]
        return jnp.maximum(h, 0.0)

    h = layer(combined, w1_ref, b1_ref, g1_ref, be1_ref)
    out_ref[...] = layer(h, w2_ref, b2_ref, g2_ref, be2_ref)


def _mlp(x, parts, W1, b1, g1, be1, W2, b2, g2, be2):
    vecs = [v.reshape(1, _I) for v in (b1, g1, be1, b2, g2, be2)]
    return pl.pallas_call(
        _mlp_body,
        out_shape=jax.ShapeDtypeStruct((_N, _I), jnp.float32),
    )(x, parts, W1, vecs[0], vecs[1], vecs[2], W2, vecs[3], vecs[4], vecs[5])


def kernel(node_embeddings, edge_index, edge_weights,
           W1, b1, g1, be1, W2, b2, g2, be2, hop_coef):
    x = node_embeddings
    table, idx, srcf = _build_table_and_idx(
        x, jnp.broadcast_to(hop_coef[:, None, None], (_D, 1, _I)),
        edge_weights.reshape(_E // _I, _I),
        edge_index[1].reshape(_E // _I, _I),
        edge_index[0].reshape(_E // _I, _I))
    parts = _sc_aggregate(table, idx.reshape(_E), srcf.reshape(_E))
    return _mlp(x, parts.reshape(_NC, _N, _I),
                W1, b1, g1, be1, W2, b2, g2, be2)


# speedup vs baseline: 21.3735x; 21.3735x over previous
import functools

import jax
import jax.numpy as jnp
from jax import lax
from jax.experimental import pallas as pl
from jax.experimental.pallas import tpu as pltpu
from jax.experimental.pallas import tpu_sc as plsc

_N, _E, _I, _D = 10000, 320000, 128, 3
_NC, _NS = 2, 16
_NW = _NC * _NS
_EPW = _E // _NW
_C = 80
_NCH = _EPW // _C
_UROWS = 80
_NU = _N // _UROWS


def _prep_body(coef_ref, x_ref, w_ref, dst_ref, src_ref,
               table_ref, idx_ref, srcf_ref):
    table_ref[...] = x_ref[...] * coef_ref[0]

    @pl.when(pl.program_id(0) == 0)
    def _():
        w = w_ref[...]
        idx_ref[...] = jnp.maximum(w - 1, 0) * _N + dst_ref[...]
        rows = lax.broadcasted_iota(jnp.int32, w.shape, 0)
        cols = lax.broadcasted_iota(jnp.int32, w.shape, 1)
        trash = _N + lax.rem(rows * _I + cols, _UROWS)
        srcf_ref[...] = jnp.where(w > 0, src_ref[...], trash)


def _build_table_and_idx(x, coefs, w2d, dst2d, src2d):
    return pl.pallas_call(
        _prep_body,
        grid=(_D,),
        in_specs=[
            pl.BlockSpec((1, 1, _I), lambda d: (d, 0, 0)),
            pl.BlockSpec((_N, _I), lambda d: (0, 0)),
            pl.BlockSpec(w2d.shape, lambda d: (0, 0)),
            pl.BlockSpec(w2d.shape, lambda d: (0, 0)),
            pl.BlockSpec(w2d.shape, lambda d: (0, 0)),
        ],
        out_specs=[
            pl.BlockSpec((_N, _I), lambda d: (d, 0)),
            pl.BlockSpec(w2d.shape, lambda d: (0, 0)),
            pl.BlockSpec(w2d.shape, lambda d: (0, 0)),
        ],
        out_shape=[
            jax.ShapeDtypeStruct((_D * _N, _I), jnp.float32),
            jax.ShapeDtypeStruct(w2d.shape, jnp.int32),
            jax.ShapeDtypeStruct(w2d.shape, jnp.int32),
        ],
    )(coefs, x, w2d, dst2d, src2d)


def _sc_aggregate(table, idx, src):
    mesh = plsc.VectorSubcoreMesh(core_axis_name="c", subcore_axis_name="s")

    @functools.partial(
        pl.kernel,
        out_type=jax.ShapeDtypeStruct((_NC * _N, _I), jnp.float32),
        mesh=mesh,
        scratch_types=[
            pltpu.VMEM((_EPW,), jnp.int32),
            pltpu.VMEM((_C,), jnp.int32),
            pltpu.VMEM((_C,), jnp.int32),
            pltpu.VMEM((_C,), jnp.int32),
            pltpu.VMEM((_C, _I), jnp.float32),
            pltpu.VMEM((_C, _I), jnp.float32),
            pltpu.VMEM((_C, _I), jnp.float32),
            pltpu.VMEM((_UROWS // 2, _I), jnp.float32),
            pltpu.VMEM_SHARED((_N + _UROWS, _I), jnp.float32),
            pltpu.SemaphoreType.DMA,
            pltpu.SemaphoreType.DMA,
            pltpu.SemaphoreType.DMA,
            pltpu.SemaphoreType.DMA,
            pltpu.SemaphoreType.DMA,
            pltpu.SemaphoreType.DMA,
            pltpu.SemaphoreType.DMA,
        ],
    )
    def body(table_hbm, idx_hbm, src_hbm, out_hbm,
             idx_buf, src_0, src_1, src_2, bank_0, bank_1, bank_2, zbuf,
             accum, fsem_0, fsem_1, fsem_2, ssem_0, ssem_1, ssem_2, isem):
        srcs = (src_0, src_1, src_2)
        banks = (bank_0, bank_1, bank_2)
        fsems = (fsem_0, fsem_1, fsem_2)
        ssems = (ssem_0, ssem_1, ssem_2)
        c = lax.axis_index("c")
        s = lax.axis_index("s")
        wid = s * _NC + c
        base = pl.multiple_of(wid * _EPW, 8)
        pltpu.sync_copy(idx_hbm.at[pl.ds(base, _EPW)], idx_buf)

        n_units = (_NU - 1 - s) // _NS + 1

        def init_unit(k, carry):
            r = pl.multiple_of((s + k * _NS) * _UROWS, 8)
            lo = pltpu.make_async_copy(
                zbuf, accum.at[pl.ds(r, _UROWS // 2)], isem)
            hi = pltpu.make_async_copy(
                zbuf, accum.at[pl.ds(r + _UROWS // 2, _UROWS // 2)], isem)
            return lo, hi

        def fetch(ch, r):
            off = pl.multiple_of(ch * _C, 8)
            rows = pltpu.make_async_copy(
                table_hbm.at[idx_buf.at[pl.ds(off, _C)]], banks[r], fsems[r])
            sidx = pltpu.make_async_copy(
                src_hbm.at[pl.ds(base + off, _C)], srcs[r], fsems[r])
            return rows, sidx

        def scat(r):
            return pltpu.make_async_copy(banks[r], accum.at[srcs[r]], ssems[r])

        for cp in fetch(0, 0) + fetch(1, 1):
            cp.start()

        def zero_row(i, carry):
            for j in range(_I // 16):
                zbuf[i, pl.ds(j * 16, 16)] = jnp.zeros((16,), jnp.float32)
            return carry

        lax.fori_loop(0, _UROWS // 2, zero_row, 0)

        def start_init(k, carry):
            for cp in init_unit(k, carry):
                cp.start()
            return carry

        def wait_init(k, carry):
            for cp in init_unit(k, carry):
                cp.wait()
            return carry

        lax.fori_loop(0, n_units, start_init, 0)
        lax.fori_loop(0, n_units, wait_init, 0)
        plsc.subcore_barrier()

        def run_chunk(ch, r):
            t = (r + 2) % 3

            @pl.when((ch >= 1) & (ch < _NCH - 2))
            def _():
                scat(t).wait()

            @pl.when(ch < _NCH - 2)
            def _():
                for cp in fetch(ch + 2, t):
                    cp.start()

            for cp in fetch(ch, r):
                cp.wait()
            scat(r).start(add=True)

        def triple_body(g, carry):
            for r in range(3):
                run_chunk(g * 3 + r, r)
            return carry

        lax.fori_loop(0, _NCH // 3, triple_body, 0)
        for ch in range(_NCH - _NCH % 3, _NCH):
            for cp in fetch(ch, ch % 3):
                cp.wait()
            scat(ch % 3).start(add=True)
        for r in ((_NCH - 3) % 3, (_NCH - 2) % 3, (_NCH - 1) % 3):
            scat(r).wait()
        plsc.subcore_barrier()

        def write_unit(k, carry):
            r = pl.multiple_of((s + k * _NS) * _UROWS, 8)
            return pltpu.make_async_copy(accum.at[pl.ds(r, _UROWS)],
                                         out_hbm.at[pl.ds(c * _N + r, _UROWS)],
                                         isem)

        lax.fori_loop(0, n_units,
                      lambda k, c: (write_unit(k, c).start(), c)[1], 0)
        lax.fori_loop(0, n_units,
                      lambda k, c: (write_unit(k, c).wait(), c)[1], 0)

    return body(table, idx, src)


def _mlp_body(x_ref, parts_ref, w1_ref, b1_ref, g1_ref, be1_ref,
              w2_ref, b2_ref, g2_ref, be2_ref, out_ref):
    combined = x_ref[...] + parts_ref[0] + parts_ref[1]

    def layer(h, w_ref, b_ref, g_ref, be_ref):
        h = lax.dot_general(h, w_ref[...], (((1,), (1,)), ((), ())),
                            preferred_element_type=jnp.float32)
        h = h + b_ref[...]
        mu = jnp.mean(h, axis=0, keepdims=True)
        var = jnp.mean((h - mu) ** 2, axis=0, keepdims=True)
        h = g_ref[...] * (h - mu) / jnp.sqrt(var + 1e-5) + be_ref[...]
        return jnp.maximum(h, 0.0)

    h = layer(combined, w1_ref, b1_ref, g1_ref, be1_ref)
    out_ref[...] = layer(h, w2_ref, b2_ref, g2_ref, be2_ref)


def _mlp(x, parts, W1, b1, g1, be1, W2, b2, g2, be2):
    vecs = [v.reshape(1, _I) for v in (b1, g1, be1, b2, g2, be2)]
    return pl.pallas_call(
        _mlp_body,
        out_shape=jax.ShapeDtypeStruct((_N, _I), jnp.float32),
    )(x, parts, W1, vecs[0], vecs[1], vecs[2], W2, vecs[3], vecs[4], vecs[5])


def kernel(node_embeddings, edge_index, edge_weights,
           W1, b1, g1, be1, W2, b2, g2, be2, hop_coef):
    x = node_embeddings
    table, idx, srcf = _build_table_and_idx(
        x, jnp.broadcast_to(hop_coef[:, None, None], (_D, 1, _I)),
        edge_weights.reshape(_E // _I, _I),
        edge_index[1].reshape(_E // _I, _I),
        edge_index[0].reshape(_E // _I, _I))
    parts = _sc_aggregate(table, idx.reshape(_E), srcf.reshape(_E))
    return _mlp(x, parts.reshape(_NC, _N, _I),
                W1, b1, g1, be1, W2, b2, g2, be2)
